# baseline (speedup 1.0000x reference)
import functools

import jax
import jax.numpy as jnp
from jax import lax
from jax.experimental import pallas as pl
from jax.experimental.pallas import tpu as pltpu
from jax.experimental.pallas import tpu_sc as plsc


@functools.lru_cache(maxsize=None)
def _make_gather(n_tab, d, h, w, resolution):
    info = plsc.get_sparse_core_info()
    nc, ns = info.num_cores, info.num_subcores
    nw = nc * ns
    n_sp = h * w
    n_out = n_sp + 1

    br = max(b for b in range(1, w + 1) if w % b == 0 and (n_sp // b) % nw == 0)
    bpw = n_sp // br // nw
    bpr = w // br

    mesh = plsc.VectorSubcoreMesh(core_axis_name="c", subcore_axis_name="s")

    @functools.partial(
        pl.kernel,
        mesh=mesh,
        out_type=jax.ShapeDtypeStruct((n_out, d), jnp.float32),
        scratch_types=[
            pltpu.VMEM((bpw, br, d), jnp.float32),
            pltpu.VMEM((1, d), jnp.float32),
            pltpu.SemaphoreType.DMA((bpw,)),
            pltpu.SemaphoreType.DMA((bpw,)),
            pltpu.SemaphoreType.DMA,
        ],
        compiler_params=pltpu.CompilerParams(use_tc_tiling_on_sc=False),
    )
    def gather_kernel(table_hbm, out_hbm, bufs, cls_v, gsem, ssem, csem):
        wid = lax.axis_index("s") * nc + lax.axis_index("c")

        copies = []
        for b in range(bpw):
            blk = wid * bpw + b
            src = (blk // bpr) * resolution + (blk % bpr) * br + 1
            cp = pltpu.make_async_copy(
                table_hbm.at[pl.ds(src, br)],
                out_hbm.at[pl.ds(blk * br + 1, br)],
                gsem.at[b],
            )
            cp.start()
            copies.append(cp)

        @pl.when(wid == 0)
        def _copy_cls():
            pltpu.make_async_copy(
                table_hbm.at[pl.ds(0, 1)], out_hbm.at[pl.ds(0, 1)], csem
            ).start()

        for cp in copies:
            cp.wait()

        @pl.when(wid == 0)
        def _wait_cls():
            pltpu.make_async_copy(
                table_hbm.at[pl.ds(0, 1)], out_hbm.at[pl.ds(0, 1)], csem
            ).wait()

    def run(vpe):
        return gather_kernel(vpe)

    return run


def kernel(x, vpe):
    resolution = round((vpe.shape[0] - 1) ** 0.5)
    assert resolution * resolution + 1 == vpe.shape[0]
    _, _, h, w = x.shape
    return _make_gather(vpe.shape[0], vpe.shape[1], h, w, resolution)(vpe)


# speedup vs baseline: 3.5575x; 3.5575x over previous
import functools

import jax
import jax.numpy as jnp
from jax import lax
from jax.experimental import pallas as pl
from jax.experimental.pallas import tpu as pltpu
from jax.experimental.pallas import tpu_sc as plsc


@functools.lru_cache(maxsize=None)
def _make_gather(n_tab, d, h, w, resolution):
    info = plsc.get_sparse_core_info()
    nc, ns = info.num_cores, info.num_subcores
    nw = nc * ns
    n_sp = h * w
    n_out = n_sp + 1

    br = max(b for b in range(1, w + 1) if w % b == 0 and (n_sp // b) % nw == 0)
    bpw = n_sp // br // nw
    bpr = w // br

    mesh = plsc.VectorSubcoreMesh(core_axis_name="c", subcore_axis_name="s")

    @functools.partial(
        pl.kernel,
        mesh=mesh,
        out_type=jax.ShapeDtypeStruct((n_out, d), jnp.float32),
        scratch_types=[
            pltpu.VMEM((bpw, br, d), jnp.float32),
            pltpu.VMEM((1, d), jnp.float32),
            pltpu.SemaphoreType.DMA((bpw,)),
            pltpu.SemaphoreType.DMA((bpw,)),
            pltpu.SemaphoreType.DMA,
        ],
        compiler_params=pltpu.CompilerParams(
            use_tc_tiling_on_sc=False,
            disable_bounds_checks=True,
            disable_semaphore_checks=True,
        ),
    )
    def gather_kernel(table_hbm, out_hbm, bufs, cls_v, gsem, ssem, csem):
        wid = lax.axis_index("s") * nc + lax.axis_index("c")

        gathers = []
        for b in range(bpw):
            blk = wid * bpw + b
            src = (blk // bpr) * resolution + (blk % bpr) * br + 1
            cp = pltpu.make_async_copy(
                table_hbm.at[pl.ds(src, br)], bufs.at[b], gsem.at[b]
            )
            cp.start()
            gathers.append(cp)

        @pl.when(wid == 0)
        def _start_cls():
            pltpu.make_async_copy(table_hbm.at[pl.ds(0, 1)], cls_v, csem).start()

        scatters = []
        for b in range(bpw):
            blk = wid * bpw + b
            gathers[b].wait()
            cp = pltpu.make_async_copy(
                bufs.at[b], out_hbm.at[pl.ds(blk * br + 1, br)], ssem.at[b]
            )
            cp.start()
            scatters.append(cp)

        @pl.when(wid == 0)
        def _finish_cls():
            pltpu.make_async_copy(table_hbm.at[pl.ds(0, 1)], cls_v, csem).wait()
            pltpu.sync_copy(cls_v, out_hbm.at[pl.ds(0, 1)])

        for cp in scatters:
            cp.wait()

    def run(vpe):
        return gather_kernel(vpe)

    return run


def kernel(x, vpe):
    resolution = round((vpe.shape[0] - 1) ** 0.5)
    assert resolution * resolution + 1 == vpe.shape[0]
    _, _, h, w = x.shape
    return _make_gather(vpe.shape[0], vpe.shape[1], h, w, resolution)(vpe)


# speedup vs baseline: 3.5590x; 1.0004x over previous
import functools

import jax
import jax.numpy as jnp
from jax import lax
from jax.experimental import pallas as pl
from jax.experimental.pallas import tpu as pltpu
from jax.experimental.pallas import tpu_sc as plsc


@functools.lru_cache(maxsize=None)
def _make_gather(n_tab, d, h, w, resolution):
    info = plsc.get_sparse_core_info()
    nc, ns = info.num_cores, info.num_subcores
    nw = nc * ns
    n_sp = h * w
    n_out = n_sp + 1

    br = max(b for b in range(1, w + 1) if w % b == 0 and (n_sp // b) % nw == 0)
    bpw = n_sp // br // nw
    bpr = w // br

    mesh = plsc.VectorSubcoreMesh(core_axis_name="c", subcore_axis_name="s")

    @functools.partial(
        pl.kernel,
        mesh=mesh,
        out_type=jax.ShapeDtypeStruct((n_out, d), jnp.float32),
        scratch_types=[
            pltpu.VMEM((bpw * br, d), jnp.float32),
            pltpu.VMEM((1, d), jnp.float32),
            pltpu.SemaphoreType.DMA((bpw,)),
            pltpu.SemaphoreType.DMA,
            pltpu.SemaphoreType.DMA,
        ],
        compiler_params=pltpu.CompilerParams(use_tc_tiling_on_sc=False),
    )
    def gather_kernel(table_hbm, out_hbm, bufs, cls_v, gsem, ssem, csem):
        wid = lax.axis_index("s") * nc + lax.axis_index("c")

        gathers = []
        for b in range(bpw):
            blk = wid * bpw + b
            src = (blk // bpr) * resolution + (blk % bpr) * br + 1
            cp = pltpu.make_async_copy(
                table_hbm.at[pl.ds(src, br)],
                bufs.at[pl.ds(b * br, br)],
                gsem.at[b],
            )
            cp.start()
            gathers.append(cp)

        @pl.when(wid == 0)
        def _start_cls():
            pltpu.make_async_copy(table_hbm.at[pl.ds(0, 1)], cls_v, csem).start()

        for cp in gathers:
            cp.wait()
        scatter = pltpu.make_async_copy(
            bufs, out_hbm.at[pl.ds(wid * bpw * br + 1, bpw * br)], ssem
        )
        scatter.start()

        @pl.when(wid == 0)
        def _finish_cls():
            pltpu.make_async_copy(table_hbm.at[pl.ds(0, 1)], cls_v, csem).wait()
            pltpu.sync_copy(cls_v, out_hbm.at[pl.ds(0, 1)])

        scatter.wait()

    def run(vpe):
        return gather_kernel(vpe)

    return run


def kernel(x, vpe):
    resolution = round((vpe.shape[0] - 1) ** 0.5)
    assert resolution * resolution + 1 == vpe.shape[0]
    _, _, h, w = x.shape
    return _make_gather(vpe.shape[0], vpe.shape[1], h, w, resolution)(vpe)
